# Initial kernel scaffold; baseline (speedup 1.0000x reference)
import jax
import jax.numpy as jnp
from jax.experimental import pallas as pl


def kernel(s, emb_w, lin_w, lin_b):
    raise NotImplementedError("write your pallas kernel here")


# speedup vs baseline: 27.7583x; 27.7583x over previous
import functools

import jax
import jax.numpy as jnp
from jax import lax
from jax.experimental import pallas as pl
from jax.experimental.pallas import tpu as pltpu
from jax.experimental.pallas import tpu_sc as plsc

VOCAB = 100000
EMBED_DIM = 128
BATCH = 16384
HIST_LEN = 50
OUT_DIM = 2

VBLK = 4000
NLANE = 16
NWORK = 32
NCHUNK_B = NWORK // OUT_DIM
B_PER_CHUNK = BATCH // NCHUNK_B
NSUB = 2
B_SUB = B_PER_CHUNK // NSUB
NGRP = B_SUB // NLANE


def _proj_body(w_ref, b_ref, x_ref, out_ref):
    y = lax.dot_general(
        x_ref[...], w_ref[...],
        dimension_numbers=(((1,), (1,)), ((), ())),
        preferred_element_type=jnp.float32,
    )
    out_ref[...] = y + b_ref[...]


def _project(emb_w, lin_w, lin_b):
    return pl.pallas_call(
        _proj_body,
        grid=(VOCAB // VBLK,),
        in_specs=[
            pl.BlockSpec((OUT_DIM, EMBED_DIM), lambda i: (0, 0)),
            pl.BlockSpec((1, OUT_DIM), lambda i: (0, 0)),
            pl.BlockSpec((VBLK, EMBED_DIM), lambda i: (i, 0)),
        ],
        out_specs=pl.BlockSpec((VBLK, OUT_DIM), lambda i: (i, 0)),
        out_shape=jax.ShapeDtypeStruct((VOCAB, OUT_DIM), jnp.float32),
    )(lin_w, lin_b.reshape(1, OUT_DIM), emb_w)


def _sc_body(p_hbm, idx_hbm, out_hbm, p_v, idx_v, out_v):
    cid_axis = lax.axis_index("c")
    sid_axis = lax.axis_index("s")
    wid = sid_axis * 2 + cid_axis
    col = wid % OUT_DIM
    chunk = wid // OUT_DIM

    pltpu.sync_copy(p_hbm.at[col], p_v)

    lane = lax.iota(jnp.int32, NLANE)
    lane_base = lane * HIST_LEN

    for sub in range(NSUB):
        pltpu.sync_copy(idx_hbm.at[chunk, sub], idx_v)

        def group_body(g, _):
            base = lane_base + g * (NLANE * HIST_LEN)
            acc = jnp.zeros((NLANE,), jnp.float32)
            for l in range(HIST_LEN):
                idx = plsc.load_gather(idx_v, [base + l])
                acc = acc + plsc.load_gather(p_v, [idx])
            out_v[g, :] = acc
            return 0

        lax.fori_loop(0, NGRP, group_body, 0)
        pltpu.sync_copy(out_v, out_hbm.at[col, chunk, sub])


def _gather_sum(p_t, s):
    mesh = plsc.VectorSubcoreMesh(core_axis_name="c", subcore_axis_name="s")
    k = functools.partial(
        pl.kernel,
        out_type=jax.ShapeDtypeStruct(
            (OUT_DIM, NCHUNK_B, NSUB, NGRP, NLANE), jnp.float32),
        mesh=mesh,
        scratch_types=[
            pltpu.VMEM((VOCAB,), jnp.float32),
            pltpu.VMEM((B_SUB * HIST_LEN,), jnp.int32),
            pltpu.VMEM((NGRP, NLANE), jnp.float32),
        ],
        compiler_params=pltpu.CompilerParams(needs_layout_passes=False),
    )(_sc_body)
    idx = s.reshape(NCHUNK_B, NSUB, B_SUB * HIST_LEN)
    return k(p_t, idx)


def kernel(s, emb_w, lin_w, lin_b):
    p_t = _project(emb_w, lin_w, lin_b).T
    out = _gather_sum(p_t, s.astype(jnp.int32))
    return out.reshape(OUT_DIM, BATCH).T


# speedup vs baseline: 34.6707x; 1.2490x over previous
import functools

import jax
import jax.numpy as jnp
from jax import lax
from jax.experimental import pallas as pl
from jax.experimental.pallas import tpu as pltpu
from jax.experimental.pallas import tpu_sc as plsc

VOCAB = 100000
EMBED_DIM = 128
BATCH = 16384
HIST_LEN = 50
OUT_DIM = 2

VBLK = 2048
VPAD = 100352
NLANE = 16
NWORK = 32
NCHUNK_B = NWORK // OUT_DIM
B_PER_CHUNK = BATCH // NCHUNK_B
NSUB = 2
B_SUB = B_PER_CHUNK // NSUB
NGRP = B_SUB // NLANE


def _proj_body(w_ref, b_ref, x_ref, out_ref):
    y = lax.dot_general(
        w_ref[...], x_ref[...],
        dimension_numbers=(((1,), (1,)), ((), ())),
        preferred_element_type=jnp.float32,
    )
    out_ref[...] = y + b_ref[...]


def _project(emb_w, lin_w, lin_b):
    return pl.pallas_call(
        _proj_body,
        grid=(VPAD // VBLK,),
        in_specs=[
            pl.BlockSpec((OUT_DIM, EMBED_DIM), lambda i: (0, 0)),
            pl.BlockSpec((OUT_DIM, 1), lambda i: (0, 0)),
            pl.BlockSpec((VBLK, EMBED_DIM), lambda i: (i, 0)),
        ],
        out_specs=pl.BlockSpec((OUT_DIM, VBLK), lambda i: (0, i)),
        out_shape=jax.ShapeDtypeStruct((OUT_DIM, VPAD), jnp.float32),
    )(lin_w, lin_b.reshape(OUT_DIM, 1), emb_w)


def _sc_body(p_hbm, idx_hbm, out_hbm, p_v, idx_v, out_v):
    cid_axis = lax.axis_index("c")
    sid_axis = lax.axis_index("s")
    wid = sid_axis * 2 + cid_axis
    col = wid % OUT_DIM
    chunk = wid // OUT_DIM

    pltpu.sync_copy(p_hbm.at[col], p_v)

    lane = lax.iota(jnp.int32, NLANE)
    lane_base = lane * HIST_LEN

    for sub in range(NSUB):
        pltpu.sync_copy(idx_hbm.at[chunk, sub], idx_v)

        def group_body(g, _):
            base = lane_base + g * (NLANE * HIST_LEN)
            acc = jnp.zeros((NLANE,), jnp.float32)
            for l in range(HIST_LEN):
                idx = plsc.load_gather(idx_v, [base + l])
                acc = acc + plsc.load_gather(p_v, [idx])
            out_v[g, :] = acc
            return 0

        lax.fori_loop(0, NGRP, group_body, 0)
        pltpu.sync_copy(out_v, out_hbm.at[col, chunk, sub])


def _gather_sum(p_t, s):
    mesh = plsc.VectorSubcoreMesh(core_axis_name="c", subcore_axis_name="s")
    k = functools.partial(
        pl.kernel,
        out_type=jax.ShapeDtypeStruct(
            (OUT_DIM, NCHUNK_B, NSUB, NGRP, NLANE), jnp.float32),
        mesh=mesh,
        scratch_types=[
            pltpu.VMEM((VPAD,), jnp.float32),
            pltpu.VMEM((B_SUB * HIST_LEN,), jnp.int32),
            pltpu.VMEM((NGRP, NLANE), jnp.float32),
        ],
        compiler_params=pltpu.CompilerParams(needs_layout_passes=False),
    )(_sc_body)
    idx = s.reshape(NCHUNK_B, NSUB, B_SUB * HIST_LEN)
    return k(p_t, idx)


def kernel(s, emb_w, lin_w, lin_b):
    p_t = _project(emb_w, lin_w, lin_b)
    out = _gather_sum(p_t, s.astype(jnp.int32))
    return out.reshape(OUT_DIM, BATCH).T


# speedup vs baseline: 35.0031x; 1.0096x over previous
import functools

import jax
import jax.numpy as jnp
from jax import lax
from jax.experimental import pallas as pl
from jax.experimental.pallas import tpu as pltpu
from jax.experimental.pallas import tpu_sc as plsc

VOCAB = 100000
EMBED_DIM = 128
BATCH = 16384
HIST_LEN = 50
OUT_DIM = 2

VBLK = 2048
VPAD = 100352
NLANE = 16
NWORK = 32
B_W = BATCH // NWORK
NGRP = B_W // NLANE
NSUB = 2
B_SUB = B_W // NSUB
NGRP_SUB = B_SUB // NLANE


def _proj_body(w_ref, b_ref, x_ref, out_ref):
    y = lax.dot_general(
        w_ref[...], x_ref[...],
        dimension_numbers=(((1,), (1,)), ((), ())),
        preferred_element_type=jnp.float32,
    )
    y = y + b_ref[...]
    u = lax.bitcast_convert_type(y, jnp.uint32)
    r = (u + jnp.uint32(0x7FFF) + ((u >> 16) & jnp.uint32(1))) >> 16
    packed = r[0:1, :] | (r[1:2, :] << 16)
    out_ref[...] = lax.bitcast_convert_type(packed.reshape(VBLK), jnp.int32)


def _project(emb_w, lin_w, lin_b):
    return pl.pallas_call(
        _proj_body,
        grid=(VPAD // VBLK,),
        in_specs=[
            pl.BlockSpec((OUT_DIM, EMBED_DIM), lambda i: (0, 0)),
            pl.BlockSpec((OUT_DIM, 1), lambda i: (0, 0)),
            pl.BlockSpec((VBLK, EMBED_DIM), lambda i: (i, 0)),
        ],
        out_specs=pl.BlockSpec((VBLK,), lambda i: (i,)),
        out_shape=jax.ShapeDtypeStruct((VPAD,), jnp.int32),
    )(lin_w, lin_b.reshape(OUT_DIM, 1), emb_w)


def _sc_body(p_hbm, idx_hbm, out_hbm, p_v, idx_v, out_v):
    cid_axis = lax.axis_index("c")
    sid_axis = lax.axis_index("s")
    wid = sid_axis * 2 + cid_axis

    pltpu.sync_copy(p_hbm, p_v)

    lane = lax.iota(jnp.int32, NLANE)
    hi_mask = jnp.full((NLANE,), -65536, jnp.int32)

    for sub in range(NSUB):
        pltpu.sync_copy(idx_hbm.at[wid, sub], idx_v)

        def group_body(g, _, sub=sub):
            base = (g * NLANE + lane) * HIST_LEN
            acc0 = jnp.zeros((NLANE,), jnp.float32)
            acc1 = jnp.zeros((NLANE,), jnp.float32)
            for l in range(HIST_LEN):
                idx = plsc.load_gather(idx_v, [base + l])
                packed = plsc.load_gather(p_v, [idx])
                f0 = plsc.bitcast(packed << 16, jnp.float32)
                f1 = plsc.bitcast(packed & hi_mask, jnp.float32)
                acc0 = acc0 + f0
                acc1 = acc1 + f1
            out_v[0, sub * NGRP_SUB + g, :] = acc0
            out_v[1, sub * NGRP_SUB + g, :] = acc1
            return 0

        lax.fori_loop(0, NGRP_SUB, group_body, 0)

    pltpu.sync_copy(out_v, out_hbm.at[wid])


def _gather_sum(p_packed, s):
    mesh = plsc.VectorSubcoreMesh(core_axis_name="c", subcore_axis_name="s")
    k = functools.partial(
        pl.kernel,
        out_type=jax.ShapeDtypeStruct(
            (NWORK, OUT_DIM, NGRP, NLANE), jnp.float32),
        mesh=mesh,
        scratch_types=[
            pltpu.VMEM((VPAD,), jnp.int32),
            pltpu.VMEM((B_SUB * HIST_LEN,), jnp.int32),
            pltpu.VMEM((OUT_DIM, NGRP, NLANE), jnp.float32),
        ],
        compiler_params=pltpu.CompilerParams(needs_layout_passes=False),
    )(_sc_body)
    return k(p_packed, s.reshape(NWORK, NSUB, B_SUB * HIST_LEN))


def kernel(s, emb_w, lin_w, lin_b):
    p_packed = _project(emb_w, lin_w, lin_b)
    out = _gather_sum(p_packed, s.astype(jnp.int32))
    return out.reshape(NWORK, OUT_DIM, B_W).transpose(0, 2, 1).reshape(
        BATCH, OUT_DIM)


# speedup vs baseline: 35.4754x; 1.0135x over previous
import functools

import jax
import jax.numpy as jnp
from jax import lax
from jax.experimental import pallas as pl
from jax.experimental.pallas import tpu as pltpu
from jax.experimental.pallas import tpu_sc as plsc

VOCAB = 100000
EMBED_DIM = 128
BATCH = 16384
HIST_LEN = 50
OUT_DIM = 2

VBLK = 2048
VPAD = 100352
NLANE = 16
NWORK = 32
B_W = BATCH // NWORK
NGRP = B_W // NLANE
NSUB = 4
B_SUB = B_W // NSUB
NGRP_SUB = B_SUB // NLANE


def _proj_body(w_ref, b_ref, x_ref, out_ref):
    y = lax.dot_general(
        w_ref[...], x_ref[...],
        dimension_numbers=(((1,), (1,)), ((), ())),
        preferred_element_type=jnp.float32,
    )
    y = y + b_ref[...]
    u = lax.bitcast_convert_type(y, jnp.uint32)
    r = (u + jnp.uint32(0x7FFF) + ((u >> 16) & jnp.uint32(1))) >> 16
    packed = r[0:1, :] | (r[1:2, :] << 16)
    out_ref[...] = lax.bitcast_convert_type(packed.reshape(VBLK), jnp.int32)


def _project(emb_w, lin_w, lin_b):
    return pl.pallas_call(
        _proj_body,
        grid=(VPAD // VBLK,),
        in_specs=[
            pl.BlockSpec((OUT_DIM, EMBED_DIM), lambda i: (0, 0)),
            pl.BlockSpec((OUT_DIM, 1), lambda i: (0, 0)),
            pl.BlockSpec((VBLK, EMBED_DIM), lambda i: (i, 0)),
        ],
        out_specs=pl.BlockSpec((VBLK,), lambda i: (i,)),
        out_shape=jax.ShapeDtypeStruct((VPAD,), jnp.int32),
    )(lin_w, lin_b.reshape(OUT_DIM, 1), emb_w)


def _sc_body(p_hbm, idx_hbm, out_hbm, p_v, idx_v0, idx_v1, out_v,
             sem_p, sem_i0, sem_i1):
    cid_axis = lax.axis_index("c")
    sid_axis = lax.axis_index("s")
    wid = sid_axis * 2 + cid_axis
    base_s = wid * (B_W * HIST_LEN)

    idx_bufs = (idx_v0, idx_v1)
    sems = (sem_i0, sem_i1)

    cp_p = pltpu.async_copy(p_hbm, p_v, sem_p)
    cp = [None, None]
    cp[0] = pltpu.async_copy(
        idx_hbm.at[pl.ds(base_s, B_SUB * HIST_LEN)], idx_v0, sem_i0)
    cp_p.wait()

    lane = lax.iota(jnp.int32, NLANE)
    hi_mask = jnp.full((NLANE,), -65536, jnp.int32)

    for sub in range(NSUB):
        buf = idx_bufs[sub % 2]
        cp[sub % 2].wait()
        if sub + 1 < NSUB:
            cp[(sub + 1) % 2] = pltpu.async_copy(
                idx_hbm.at[pl.ds(base_s + (sub + 1) * B_SUB * HIST_LEN,
                                 B_SUB * HIST_LEN)],
                idx_bufs[(sub + 1) % 2], sems[(sub + 1) % 2])

        def group_body(g, _, sub=sub, buf=buf):
            base = (g * NLANE + lane) * HIST_LEN
            acc0 = jnp.zeros((NLANE,), jnp.float32)
            acc1 = jnp.zeros((NLANE,), jnp.float32)
            for l in range(HIST_LEN):
                idx = plsc.load_gather(buf, [base + l])
                packed = plsc.load_gather(p_v, [idx])
                f0 = plsc.bitcast(packed << 16, jnp.float32)
                f1 = plsc.bitcast(packed & hi_mask, jnp.float32)
                acc0 = acc0 + f0
                acc1 = acc1 + f1
            out_v[0, sub * NGRP_SUB + g, :] = acc0
            out_v[1, sub * NGRP_SUB + g, :] = acc1
            return 0

        lax.fori_loop(0, NGRP_SUB, group_body, 0)

    pltpu.sync_copy(out_v, out_hbm.at[wid])


def _gather_sum(p_packed, s):
    mesh = plsc.VectorSubcoreMesh(core_axis_name="c", subcore_axis_name="s")
    k = functools.partial(
        pl.kernel,
        out_type=jax.ShapeDtypeStruct(
            (NWORK, OUT_DIM, NGRP, NLANE), jnp.float32),
        mesh=mesh,
        scratch_types=[
            pltpu.VMEM((VPAD,), jnp.int32),
            pltpu.VMEM((B_SUB * HIST_LEN,), jnp.int32),
            pltpu.VMEM((B_SUB * HIST_LEN,), jnp.int32),
            pltpu.VMEM((OUT_DIM, NGRP, NLANE), jnp.float32),
            pltpu.SemaphoreType.DMA,
            pltpu.SemaphoreType.DMA,
            pltpu.SemaphoreType.DMA,
        ],
        compiler_params=pltpu.CompilerParams(needs_layout_passes=False),
    )(_sc_body)
    return k(p_packed, s.reshape(BATCH * HIST_LEN))


def kernel(s, emb_w, lin_w, lin_b):
    p_packed = _project(emb_w, lin_w, lin_b)
    out = _gather_sum(p_packed, s.astype(jnp.int32))
    return out.reshape(NWORK, OUT_DIM, B_W).transpose(0, 2, 1).reshape(
        BATCH, OUT_DIM)
